# Initial kernel scaffold; baseline (speedup 1.0000x reference)
import jax
import jax.numpy as jnp
from jax.experimental import pallas as pl


def kernel(n_id_performance, n_id_artist, n_id_song, edge_index, emb_performance, emb_artist, emb_song, Wl, bl, Wr, ln_gamma, ln_beta, P1_W, P1_b, P2_W, P2_b):
    raise NotImplementedError("write your pallas kernel here")


# speedup vs baseline: 2.3121x; 2.3121x over previous
import functools

import jax
import jax.numpy as jnp
from jax import lax
from jax.experimental import pallas as pl
from jax.experimental.pallas import tpu as pltpu
from jax.experimental.pallas import tpu_sc as plsc

_N = 10000
_E = 160000
_D = 128
_NC, _NS = 2, 16
_NW = _NC * _NS
_SRC = (1, 0, 1, 0, 2, 2)
_DST = (0, 2, 2, 1, 0, 1)

_EPAD = 163840
_NP = _N + 16
_PAD_DST = _N + 8
_RPT = 624

_BLK = 1000
_GRID = _N // _BLK

_mesh = plsc.VectorSubcoreMesh(
    core_axis_name="c", subcore_axis_name="s", num_cores=_NC, num_subcores=_NS)

_ECH = 128
_EMB_FULL = _N // _ECH
_EMB_TAIL = _N - _EMB_FULL * _ECH


def _embed_body(nid_p, nid_a, nid_s, emb_p, emb_a, emb_s, xp, xa, xs,
                idx_v, rows_v, idx_t, rows_t, sem):
    cid = lax.axis_index("c")
    sid = lax.axis_index("s")
    wid = cid * _NS + sid
    for t in range(3):
        nid = (nid_p, nid_a, nid_s)[t]
        emb = (emb_p, emb_a, emb_s)[t]
        out = (xp, xa, xs)[t]
        for k in range(3):
            c = wid + _NW * k

            @pl.when(c < _EMB_FULL)
            def _():
                off = c * _ECH
                pltpu.sync_copy(nid.at[pl.ds(off, _ECH)], idx_v)
                pltpu.async_copy(emb.at[idx_v], rows_v, sem).wait()
                pltpu.sync_copy(rows_v, out.at[pl.ds(off, _ECH)])

        @pl.when(wid == _NW - 1)
        def _():
            off = _EMB_FULL * _ECH
            pltpu.sync_copy(nid.at[pl.ds(off, _EMB_TAIL)], idx_t)
            pltpu.async_copy(emb.at[idx_t], rows_t, sem).wait()
            pltpu.sync_copy(rows_t, out.at[pl.ds(off, _EMB_TAIL)])


_embed = pl.kernel(
    _embed_body,
    out_type=[jax.ShapeDtypeStruct((_N, _D), jnp.float32)] * 3,
    mesh=_mesh,
    scratch_types=[
        pltpu.VMEM((_ECH,), jnp.int32),
        pltpu.VMEM((_ECH, _D), jnp.float32),
        pltpu.VMEM((_EMB_TAIL,), jnp.int32),
        pltpu.VMEM((_EMB_TAIL, _D), jnp.float32),
        pltpu.SemaphoreType.DMA,
    ],
)


def _agg_body(xp, xa, xs, ei4, zrows, part, src2, dst2, rowsA, rowsB,
              acc, gsA, gsB):
    ch, cpt = 128, 40
    cid = lax.axis_index("c")
    sid = lax.axis_index("s")
    wid = cid * _NS + sid
    s0 = cpt * wid
    r0 = sid * _RPT

    for e in range(6):
        xsrc = (xp, xa, xs)[_SRC[e]]

        for z in range(4):
            pltpu.sync_copy(zrows, acc.at[pl.ds(r0 + z * 128, 128)])

        @pl.when(sid < _NS - 1)
        def _():
            pltpu.sync_copy(zrows.at[pl.ds(0, 112)],
                            acc.at[pl.ds(r0 + 512, 112)])

        @pl.when(sid == _NS - 1)
        def _():
            pltpu.sync_copy(zrows, acc.at[pl.ds(r0 + 512, 128)])

        plsc.subcore_barrier()

        pltpu.sync_copy(ei4.at[e, 0, pl.ds(s0, cpt)], src2)
        pltpu.sync_copy(ei4.at[e, 1, pl.ds(s0, cpt)], dst2)

        def fire(k, buf, sem):
            pltpu.make_async_copy(xsrc.at[src2.at[k]], buf, sem).start()

        def drain(buf, sem):
            pltpu.make_async_copy(xsrc.at[src2.at[0]], buf, sem).wait()

        def scat(k, buf):
            pltpu.sync_copy(buf, acc.at[dst2.at[k]], add=True)

        fire(0, rowsA, gsA)

        def pair(j, carry):
            k0 = 2 * j
            fire(k0 + 1, rowsB, gsB)
            drain(rowsA, gsA)
            scat(k0, rowsA)

            @pl.when(k0 + 2 < cpt)
            def _():
                fire(k0 + 2, rowsA, gsA)

            drain(rowsB, gsB)
            scat(k0 + 1, rowsB)
            return carry

        lax.fori_loop(0, cpt // 2, pair, 0)

        plsc.subcore_barrier()

        @pl.when(sid < _NS - 1)
        def _():
            pltpu.sync_copy(acc.at[pl.ds(r0, _RPT)],
                            part.at[e, cid, pl.ds(r0, _RPT)])

        @pl.when(sid == _NS - 1)
        def _():
            pltpu.sync_copy(acc.at[pl.ds(r0, 640)],
                            part.at[e, cid, pl.ds(r0, 640)])

        plsc.subcore_barrier()


_agg = pl.kernel(
    _agg_body,
    out_type=[jax.ShapeDtypeStruct((6, _NC, _N, _D), jnp.float32)],
    mesh=_mesh,
    scratch_types=[
        pltpu.VMEM((40, 128), jnp.int32),
        pltpu.VMEM((40, 128), jnp.int32),
        pltpu.VMEM((128, _D), jnp.float32),
        pltpu.VMEM((128, _D), jnp.float32),
        pltpu.VMEM_SHARED((_NP, _D), jnp.float32),
        pltpu.SemaphoreType.DMA,
        pltpu.SemaphoreType.DMA,
    ],
)


def _cnt_body(ei4, zrows, o128, cnt, dst2, ones_v, cacc, gsA):
    cpt = 40
    cid = lax.axis_index("c")
    sid = lax.axis_index("s")
    wid = cid * _NS + sid
    s0 = cpt * wid
    r0 = sid * _RPT

    pltpu.sync_copy(o128, ones_v)

    for e in range(6):
        for z in range(4):
            pltpu.sync_copy(zrows, cacc.at[pl.ds(r0 + z * 128, 128)])

        @pl.when(sid < _NS - 1)
        def _():
            pltpu.sync_copy(zrows.at[pl.ds(0, 112)],
                            cacc.at[pl.ds(r0 + 512, 112)])

        @pl.when(sid == _NS - 1)
        def _():
            pltpu.sync_copy(zrows, cacc.at[pl.ds(r0 + 512, 128)])

        plsc.subcore_barrier()
        pltpu.sync_copy(ei4.at[e, 1, pl.ds(s0, cpt)], dst2)

        def scat_ones(k, carry):
            pltpu.sync_copy(ones_v, cacc.at[dst2.at[k]], add=True)
            return carry

        lax.fori_loop(0, cpt, scat_ones, 0)

        plsc.subcore_barrier()

        @pl.when(sid < _NS - 1)
        def _():
            pltpu.sync_copy(cacc.at[pl.ds(r0, _RPT)],
                            cnt.at[e, cid, pl.ds(r0, _RPT)])

        @pl.when(sid == _NS - 1)
        def _():
            pltpu.sync_copy(cacc.at[pl.ds(r0, 640)],
                            cnt.at[e, cid, pl.ds(r0, 640)])

        plsc.subcore_barrier()


_cnt = pl.kernel(
    _cnt_body,
    out_type=[jax.ShapeDtypeStruct((6, _NC, _N, _D), jnp.float32)],
    mesh=_mesh,
    scratch_types=[
        pltpu.VMEM((40, 128), jnp.int32),
        pltpu.VMEM((128, _D), jnp.float32),
        pltpu.VMEM_SHARED((_NP, _D), jnp.float32),
        pltpu.SemaphoreType.DMA,
    ],
)


def _recip_body(cnt_ref, orec):
    for e in range(6):
        c = cnt_ref[e, 0, :, 0] + cnt_ref[e, 1, :, 0]
        orec[e, :, 0] = 1.0 / jnp.maximum(c, 1.0)


_RBLK = 2000
_recip = pl.pallas_call(
    _recip_body,
    grid=(_N // _RBLK,),
    in_specs=[pl.BlockSpec((6, _NC, _RBLK, _D), lambda i: (0, 0, i, 0))],
    out_specs=pl.BlockSpec((6, _RBLK, 1), lambda i: (0, i, 0)),
    out_shape=jax.ShapeDtypeStruct((6, _N, 1), jnp.float32),
)


def _matT(x, w):
    return lax.dot_general(x, w, (((1,), (1,)), ((), ())),
                           preferred_element_type=jnp.float32)


def _l2norm(z):
    return z / jnp.maximum(
        jnp.sqrt(jnp.sum(z * z, axis=-1, keepdims=True)), 1e-12)


def _make_tc(last):
    def body(*refs):
        if last:
            (part, cntr, xp, xa, xs, Wl, bl, Wr,
             ln_g, ln_b, P1W, P1b, P2W, P2b, o0, o1, o2) = refs
        else:
            part, cntr, xp, xa, xs, Wl, bl, Wr, o0, o1, o2 = refs
        xin = (xp[...], xa[...], xs[...])
        outs = [None, None, None]
        for e in range(6):
            rec = cntr[e, :, 0]
            mean = (part[e, 0] + part[e, 1]) * rec[:, None]
            o = _matT(mean, Wl[e]) + bl[e][None, :] + _matT(xin[_DST[e]], Wr[e])
            o = _l2norm(o)
            outs[_DST[e]] = o if outs[_DST[e]] is None else outs[_DST[e]] + o
        for t in range(3):
            h = outs[t]
            if not last:
                (o0, o1, o2)[t][...] = jnp.maximum(h, 0.0)
            else:
                mu = jnp.mean(h, axis=-1, keepdims=True)
                var = jnp.mean((h - mu) ** 2, axis=-1, keepdims=True)
                hn = (h - mu) * lax.rsqrt(var + 1e-5) * ln_g[t][None, :] \
                    + ln_b[t][None, :]
                z = jnp.maximum(_matT(hn, P1W[t]) + P1b[t][None, :], 0.0)
                z = _matT(z, P2W[t]) + P2b[t][None, :]
                (o0, o1, o2)[t][...] = _l2norm(z)

    rowblk = lambda i: (i, 0)
    full3 = pl.BlockSpec((6, _D, _D), lambda i: (0, 0, 0))
    in_specs = [
        pl.BlockSpec((6, _NC, _BLK, _D), lambda i: (0, 0, i, 0)),
        pl.BlockSpec((6, _BLK, 1), lambda i: (0, i, 0)),
        pl.BlockSpec((_BLK, _D), rowblk),
        pl.BlockSpec((_BLK, _D), rowblk),
        pl.BlockSpec((_BLK, _D), rowblk),
        full3,
        pl.BlockSpec((6, _D), lambda i: (0, 0)),
        full3,
    ]
    if last:
        in_specs += [
            pl.BlockSpec((3, _D), lambda i: (0, 0)),
            pl.BlockSpec((3, _D), lambda i: (0, 0)),
            pl.BlockSpec((3, _D, _D), lambda i: (0, 0, 0)),
            pl.BlockSpec((3, _D), lambda i: (0, 0)),
            pl.BlockSpec((3, _D, _D), lambda i: (0, 0, 0)),
            pl.BlockSpec((3, _D), lambda i: (0, 0)),
        ]
    out_specs = [pl.BlockSpec((_BLK, _D), rowblk)] * 3
    out_shape = [jax.ShapeDtypeStruct((_N, _D), jnp.float32)] * 3
    return pl.pallas_call(
        body, grid=(_GRID,), in_specs=in_specs, out_specs=out_specs,
        out_shape=out_shape)


_tc01 = _make_tc(False)
_tc2 = _make_tc(True)


def kernel(n_id_performance, n_id_artist, n_id_song, edge_index,
           emb_performance, emb_artist, emb_song, Wl, bl, Wr,
           ln_gamma, ln_beta, P1_W, P1_b, P2_W, P2_b):
    f32 = jnp.float32
    ei = edge_index.astype(jnp.int32)
    pad = jnp.broadcast_to(
        jnp.array([0, _PAD_DST], jnp.int32)[None, :, None],
        (6, 2, _EPAD - _E))
    eip = jnp.concatenate([ei, pad], axis=2)
    ei128 = eip.reshape(6, 2, _EPAD // 128, 128)
    zrows = jnp.zeros((128, _D), f32)
    o128 = jnp.ones((128, _D), f32)

    xp, xa, xs = _embed(n_id_performance.astype(jnp.int32),
                        n_id_artist.astype(jnp.int32),
                        n_id_song.astype(jnp.int32),
                        emb_performance, emb_artist, emb_song)

    cnt128, = _cnt(ei128, zrows, o128)
    recip = _recip(cnt128)
    part, = _agg(xp, xa, xs, ei128, zrows)
    xp, xa, xs = _tc01(part, recip, xp, xa, xs, Wl[0], bl[0], Wr[0])

    part, = _agg(xp, xa, xs, ei128, zrows)
    xp, xa, xs = _tc01(part, recip, xp, xa, xs, Wl[1], bl[1], Wr[1])

    part, = _agg(xp, xa, xs, ei128, zrows)
    zp, za, zs = _tc2(part, recip, xp, xa, xs, Wl[2], bl[2], Wr[2],
                      ln_gamma, ln_beta, P1_W, P1_b, P2_W, P2_b)
    return (zp, za, zs)
